# Initial kernel scaffold; baseline (speedup 1.0000x reference)
import jax
import jax.numpy as jnp
from jax.experimental import pallas as pl


def kernel(x, lora_scale_A, lora_scale_B, lora_shift_A, lora_shift_B):
    raise NotImplementedError("write your pallas kernel here")


# speedup vs baseline: 1.9069x; 1.9069x over previous
import jax
import jax.numpy as jnp
from jax.experimental import pallas as pl
from jax.experimental.pallas import tpu as pltpu

_RANK = 4
_SCALING = 8 / 4
_EPS = 1e-5


def _ln_kernel(x_ref, sa_ref, sb_ref, ha_ref, hb_ref, o_ref):
    scale = jnp.sum(sa_ref[...] * sb_ref[...], axis=0, keepdims=True) * _SCALING
    shift = jnp.sum(ha_ref[...] * hb_ref[...], axis=0, keepdims=True) * _SCALING
    x = x_ref[...]
    mean = jnp.mean(x, axis=-1, keepdims=True)
    xc = x - mean
    var = jnp.mean(xc * xc, axis=-1, keepdims=True)
    o_ref[...] = xc * jax.lax.rsqrt(var + _EPS) * scale + shift


def kernel(x, lora_scale_A, lora_scale_B, lora_shift_A, lora_shift_B):
    B, S, N = x.shape
    rows = B * S
    x2 = x.reshape(rows, N)
    sa = lora_scale_A.T
    ha = lora_shift_A.T

    BR = 256
    lora_spec = pl.BlockSpec((_RANK, N), lambda i: (0, 0))
    out = pl.pallas_call(
        _ln_kernel,
        grid=(rows // BR,),
        in_specs=[
            pl.BlockSpec((BR, N), lambda i: (i, 0)),
            lora_spec,
            lora_spec,
            lora_spec,
            lora_spec,
        ],
        out_specs=pl.BlockSpec((BR, N), lambda i: (i, 0)),
        out_shape=jax.ShapeDtypeStruct((rows, N), x.dtype),
        compiler_params=pltpu.CompilerParams(
            dimension_semantics=("parallel",),
            vmem_limit_bytes=56 * 1024 * 1024,
        ),
    )(x2, sa, lora_scale_B, ha, lora_shift_B)
    return out.reshape(B, S, N)
